# baseline (speedup 1.0000x reference)
import functools

import jax
import jax.numpy as jnp
from jax.experimental import pallas as pl
from jax.experimental.pallas import tpu as pltpu

T = 16384
HIDDEN = 2048
E = 16
TOPK = 2
CE_ALPHA = 0.0
OVER_COMPUTE = 1.2

BT = 512


def _router_kernel(hs_ref, wg_ref, wg_ema_ref, cal_ref,
                   logits_ref, rw_ref, sel_ref, em_ref, aux_ref,
                   me_acc, ce_acc):
    i = pl.program_id(0)
    nsteps = pl.num_programs(0)

    wg_running = 0.5 * (wg_ema_ref[...] + wg_ref[...])
    lt = jnp.dot(hs_ref[...], wg_running, preferred_element_type=jnp.float32)
    logits_ref[...] = lt

    rowmax = jnp.max(lt, axis=1, keepdims=True)
    ex = jnp.exp(lt - rowmax)
    denom = jnp.sum(ex, axis=1, keepdims=True)
    probs = ex / denom

    e_row = jax.lax.broadcasted_iota(jnp.int32, (BT, E), 1)
    p1 = jnp.max(probs, axis=1, keepdims=True)
    a1 = jnp.min(jnp.where(probs == p1, e_row, E), axis=1, keepdims=True)
    oh1_b = e_row == a1
    probs_m = jnp.where(oh1_b, -1.0, probs)
    p2 = jnp.max(probs_m, axis=1, keepdims=True)
    a2 = jnp.min(jnp.where(probs_m == p2, e_row, E), axis=1, keepdims=True)

    rsum = p1 + p2
    rw_ref[...] = jnp.concatenate([p1 / rsum, p2 / rsum], axis=1)
    sel_ref[...] = jnp.concatenate([a1, a2], axis=1)

    @pl.when(i == 0)
    def _init():
        me_acc[...] = jnp.zeros_like(me_acc)
        ce_acc[...] = jnp.zeros_like(ce_acc)

    mask_b = oh1_b.astype(jnp.float32) + (e_row == a2).astype(jnp.float32)
    me_acc[...] += jnp.sum(probs, axis=0, keepdims=True)
    ce_acc[...] += jnp.sum(mask_b, axis=0, keepdims=True)

    probs_t = probs.T
    e_col = jax.lax.broadcasted_iota(jnp.int32, (E, BT), 0)
    m1 = jnp.max(probs_t, axis=0, keepdims=True)
    a1t = jnp.min(jnp.where(probs_t == m1, e_col, E), axis=0, keepdims=True)
    oh1t = e_col == a1t
    probs_tm = jnp.where(oh1t, -1.0, probs_t)
    m2 = jnp.max(probs_tm, axis=0, keepdims=True)
    a2t = jnp.min(jnp.where(probs_tm == m2, e_col, E), axis=0, keepdims=True)

    e3 = jax.lax.broadcasted_iota(jnp.int32, (E, TOPK, BT), 0)
    k3 = jax.lax.broadcasted_iota(jnp.int32, (E, TOPK, BT), 1)
    sel3 = jnp.where(k3 == 0, a1t[:, None, :], a2t[:, None, :])
    em_ref[...] = (e3 == sel3).astype(jnp.int32)

    @pl.when(i == nsteps - 1)
    def _fini():
        me = me_acc[...] / T
        ce = (ce_acc[...] / T) * cal_ref[...]
        ce_ema_new = CE_ALPHA * jnp.zeros_like(ce) + (1.0 - CE_ALPHA) * ce
        e_idx = jax.lax.broadcasted_iota(jnp.int32, (1, E), 1)
        hot_num = jnp.max(ce_ema_new)
        hot_exp = jnp.min(jnp.where(ce_ema_new == hot_num, e_idx, E))
        cold_num = jnp.min(ce_ema_new)
        cold_exp = jnp.min(jnp.where(ce_ema_new == cold_num, e_idx, E))
        me_hot = jnp.sum(jnp.where(e_idx == hot_exp, me, 0.0))
        me_cold = jnp.sum(jnp.where(e_idx == cold_exp, me, 0.0))
        aux = jnp.maximum(hot_num - cold_num * OVER_COMPUTE, 0.0)
        aux_ref[...] = (aux * (me_hot - me_cold))[None, None]


@functools.partial(jax.jit, static_argnames=())
def _run(hidden_states, wg, wg_ema, cal_weights):
    nsteps = T // BT
    cal2d = cal_weights.reshape(1, E)
    out_shapes = (
        jax.ShapeDtypeStruct((T, E), jnp.float32),
        jax.ShapeDtypeStruct((T, TOPK), jnp.float32),
        jax.ShapeDtypeStruct((T, TOPK), jnp.int32),
        jax.ShapeDtypeStruct((E, TOPK, T), jnp.int32),
        jax.ShapeDtypeStruct((1, 1), jnp.float32),
    )
    return pl.pallas_call(
        _router_kernel,
        grid=(nsteps,),
        in_specs=[
            pl.BlockSpec((BT, HIDDEN), lambda i: (i, 0)),
            pl.BlockSpec((HIDDEN, E), lambda i: (0, 0)),
            pl.BlockSpec((HIDDEN, E), lambda i: (0, 0)),
            pl.BlockSpec((1, E), lambda i: (0, 0)),
        ],
        out_specs=(
            pl.BlockSpec((BT, E), lambda i: (i, 0)),
            pl.BlockSpec((BT, TOPK), lambda i: (i, 0)),
            pl.BlockSpec((BT, TOPK), lambda i: (i, 0)),
            pl.BlockSpec((E, TOPK, BT), lambda i: (0, 0, i)),
            pl.BlockSpec((1, 1), lambda i: (0, 0)),
        ),
        out_shape=out_shapes,
        scratch_shapes=[
            pltpu.VMEM((1, E), jnp.float32),
            pltpu.VMEM((1, E), jnp.float32),
        ],
    )(hidden_states, wg, wg_ema, cal2d)


def kernel(hidden_states, output_aux_losses, wg, wg_ema, cal_weights, ce_ema):
    logits, rw, sel, em, aux = _run(
        hidden_states.astype(jnp.float32), wg, wg_ema, cal_weights)
    aux_loss = aux.reshape(()) * jnp.asarray(output_aux_losses,
                                             dtype=jnp.float32)
    return (rw.astype(hidden_states.dtype), logits, aux_loss, em, sel)


# speedup vs baseline: 1.0764x; 1.0764x over previous
import functools

import jax
import jax.numpy as jnp
from jax.experimental import pallas as pl
from jax.experimental.pallas import tpu as pltpu

T = 16384
HIDDEN = 2048
E = 16
TOPK = 2
CE_ALPHA = 0.0
OVER_COMPUTE = 1.2

BT = 512


def _router_kernel(hs_ref, wg_ref, wg_ema_ref, cal_ref,
                   logits_ref, rw_ref, sel_ref, em_ref, aux_ref,
                   me_acc, ce_acc):
    i = pl.program_id(0)
    nsteps = pl.num_programs(0)

    wg_running = 0.5 * (wg_ema_ref[...] + wg_ref[...])
    lt = jnp.dot(hs_ref[...], wg_running, preferred_element_type=jnp.float32)
    logits_ref[...] = lt

    ltt = lt.T
    cmax = jnp.max(ltt, axis=0, keepdims=True)
    ex = jnp.exp(ltt - cmax)
    den = jnp.sum(ex, axis=0, keepdims=True)
    probs_t = ex / den

    e_col = jax.lax.broadcasted_iota(jnp.int32, (E, BT), 0)
    m1 = jnp.max(probs_t, axis=0, keepdims=True)
    a1 = jnp.min(jnp.where(probs_t == m1, e_col, E), axis=0, keepdims=True)
    oh1 = e_col == a1
    probs_m = jnp.where(oh1, -1.0, probs_t)
    m2 = jnp.max(probs_m, axis=0, keepdims=True)
    a2 = jnp.min(jnp.where(probs_m == m2, e_col, E), axis=0, keepdims=True)
    oh2 = e_col == a2

    @pl.when(i == 0)
    def _init():
        me_acc[...] = jnp.zeros_like(me_acc)
        ce_acc[...] = jnp.zeros_like(ce_acc)

    mask_t = oh1.astype(jnp.float32) + oh2.astype(jnp.float32)
    me_acc[...] += jnp.sum(probs_t, axis=1, keepdims=True)
    ce_acc[...] += jnp.sum(mask_t, axis=1, keepdims=True)

    e3 = jax.lax.broadcasted_iota(jnp.int32, (E, TOPK, BT), 0)
    k3 = jax.lax.broadcasted_iota(jnp.int32, (E, TOPK, BT), 1)
    sel3 = jnp.where(k3 == 0, a1[:, None, :], a2[:, None, :])
    em_ref[...] = (e3 == sel3).astype(jnp.int32)

    rsum = m1 + m2
    pack = jnp.concatenate(
        [m1 / rsum, m2 / rsum, a1.astype(jnp.float32), a2.astype(jnp.float32),
         jnp.zeros((4, BT), jnp.float32)], axis=0)
    pack_t = pack.T
    rw_ref[...] = pack_t[:, 0:2]
    sel_ref[...] = pack_t[:, 2:4].astype(jnp.int32)

    @pl.when(i == nsteps - 1)
    def _fini():
        me = me_acc[...] / T
        ce = (ce_acc[...] / T) * cal_ref[...]
        ce_ema_new = (1.0 - CE_ALPHA) * ce
        e_idx = jax.lax.broadcasted_iota(jnp.int32, (E, 1), 0)
        hot_num = jnp.max(ce_ema_new)
        hot_exp = jnp.min(jnp.where(ce_ema_new == hot_num, e_idx, E))
        cold_num = jnp.min(ce_ema_new)
        cold_exp = jnp.min(jnp.where(ce_ema_new == cold_num, e_idx, E))
        me_hot = jnp.sum(jnp.where(e_idx == hot_exp, me, 0.0))
        me_cold = jnp.sum(jnp.where(e_idx == cold_exp, me, 0.0))
        aux = jnp.maximum(hot_num - cold_num * OVER_COMPUTE, 0.0)
        aux_ref[...] = (aux * (me_hot - me_cold))[None, None]


@jax.jit
def _run(hidden_states, wg, wg_ema, cal_weights):
    nsteps = T // BT
    cal2d = cal_weights.reshape(E, 1)
    out_shapes = (
        jax.ShapeDtypeStruct((T, E), jnp.float32),
        jax.ShapeDtypeStruct((T, TOPK), jnp.float32),
        jax.ShapeDtypeStruct((T, TOPK), jnp.int32),
        jax.ShapeDtypeStruct((E, TOPK, T), jnp.int32),
        jax.ShapeDtypeStruct((1, 1), jnp.float32),
    )
    return pl.pallas_call(
        _router_kernel,
        grid=(nsteps,),
        in_specs=[
            pl.BlockSpec((BT, HIDDEN), lambda i: (i, 0)),
            pl.BlockSpec((HIDDEN, E), lambda i: (0, 0)),
            pl.BlockSpec((HIDDEN, E), lambda i: (0, 0)),
            pl.BlockSpec((E, 1), lambda i: (0, 0)),
        ],
        out_specs=(
            pl.BlockSpec((BT, E), lambda i: (i, 0)),
            pl.BlockSpec((BT, TOPK), lambda i: (i, 0)),
            pl.BlockSpec((BT, TOPK), lambda i: (i, 0)),
            pl.BlockSpec((E, TOPK, BT), lambda i: (0, 0, i)),
            pl.BlockSpec((1, 1), lambda i: (0, 0)),
        ),
        out_shape=out_shapes,
        scratch_shapes=[
            pltpu.VMEM((E, 1), jnp.float32),
            pltpu.VMEM((E, 1), jnp.float32),
        ],
    )(hidden_states, wg, wg_ema, cal2d)


def kernel(hidden_states, output_aux_losses, wg, wg_ema, cal_weights, ce_ema):
    logits, rw, sel, em, aux = _run(
        hidden_states.astype(jnp.float32), wg, wg_ema, cal_weights)
    aux_loss = aux.reshape(()) * jnp.asarray(output_aux_losses,
                                             dtype=jnp.float32)
    return (rw.astype(hidden_states.dtype), logits, aux_loss, em, sel)


# speedup vs baseline: 1.2075x; 1.1219x over previous
import functools

import jax
import jax.numpy as jnp
from jax.experimental import pallas as pl
from jax.experimental.pallas import tpu as pltpu

T = 16384
HIDDEN = 2048
E = 16
TOPK = 2
CE_ALPHA = 0.0
OVER_COMPUTE = 1.2

BT = 1024


def _router_kernel(hs_ref, wg_ref, wg_ema_ref, cal_ref,
                   logits_ref, rw_ref, sel_ref, em_ref, aux_ref,
                   me_acc, ce_acc):
    i = pl.program_id(0)
    nsteps = pl.num_programs(0)

    wg_running = 0.5 * (wg_ema_ref[...] + wg_ref[...])
    lt = jnp.dot(hs_ref[...], wg_running, preferred_element_type=jnp.float32)
    logits_ref[...] = lt

    ltt = lt.T
    cmax = jnp.max(ltt, axis=0, keepdims=True)
    ex = jnp.exp(ltt - cmax)
    den = jnp.sum(ex, axis=0, keepdims=True)
    probs_t = ex / den

    e_col = jax.lax.broadcasted_iota(jnp.int32, (E, BT), 0)
    m1 = jnp.max(probs_t, axis=0, keepdims=True)
    a1 = jnp.min(jnp.where(probs_t == m1, e_col, E), axis=0, keepdims=True)
    oh1 = e_col == a1
    probs_m = jnp.where(oh1, -1.0, probs_t)
    m2 = jnp.max(probs_m, axis=0, keepdims=True)
    a2 = jnp.min(jnp.where(probs_m == m2, e_col, E), axis=0, keepdims=True)
    oh2 = e_col == a2

    @pl.when(i == 0)
    def _init():
        me_acc[...] = jnp.zeros_like(me_acc)
        ce_acc[...] = jnp.zeros_like(ce_acc)

    mask_t = oh1.astype(jnp.float32) + oh2.astype(jnp.float32)
    me_acc[...] += jnp.sum(probs_t, axis=1, keepdims=True)
    ce_acc[...] += jnp.sum(mask_t, axis=1, keepdims=True)

    e3 = jax.lax.broadcasted_iota(jnp.int32, (E, TOPK, BT), 0)
    k3 = jax.lax.broadcasted_iota(jnp.int32, (E, TOPK, BT), 1)
    sel3 = jnp.where(k3 == 0, a1[:, None, :], a2[:, None, :])
    em_ref[...] = (e3 == sel3).astype(jnp.int32)

    rsum = m1 + m2
    pack = jnp.concatenate(
        [m1 / rsum, m2 / rsum, a1.astype(jnp.float32), a2.astype(jnp.float32),
         jnp.zeros((4, BT), jnp.float32)], axis=0)
    pack_t = pack.T
    rw_ref[...] = pack_t[:, 0:2]
    sel_ref[...] = pack_t[:, 2:4].astype(jnp.int32)

    @pl.when(i == nsteps - 1)
    def _fini():
        me = me_acc[...] / T
        ce = (ce_acc[...] / T) * cal_ref[...]
        ce_ema_new = (1.0 - CE_ALPHA) * ce
        e_idx = jax.lax.broadcasted_iota(jnp.int32, (E, 1), 0)
        hot_num = jnp.max(ce_ema_new)
        hot_exp = jnp.min(jnp.where(ce_ema_new == hot_num, e_idx, E))
        cold_num = jnp.min(ce_ema_new)
        cold_exp = jnp.min(jnp.where(ce_ema_new == cold_num, e_idx, E))
        me_hot = jnp.sum(jnp.where(e_idx == hot_exp, me, 0.0))
        me_cold = jnp.sum(jnp.where(e_idx == cold_exp, me, 0.0))
        aux = jnp.maximum(hot_num - cold_num * OVER_COMPUTE, 0.0)
        aux_ref[...] = (aux * (me_hot - me_cold))[None, None]


@jax.jit
def _run(hidden_states, wg, wg_ema, cal_weights):
    nsteps = T // BT
    cal2d = cal_weights.reshape(E, 1)
    out_shapes = (
        jax.ShapeDtypeStruct((T, E), jnp.float32),
        jax.ShapeDtypeStruct((T, TOPK), jnp.float32),
        jax.ShapeDtypeStruct((T, TOPK), jnp.int32),
        jax.ShapeDtypeStruct((E, TOPK, T), jnp.int32),
        jax.ShapeDtypeStruct((1, 1), jnp.float32),
    )
    return pl.pallas_call(
        _router_kernel,
        grid=(nsteps,),
        in_specs=[
            pl.BlockSpec((BT, HIDDEN), lambda i: (i, 0)),
            pl.BlockSpec((HIDDEN, E), lambda i: (0, 0)),
            pl.BlockSpec((HIDDEN, E), lambda i: (0, 0)),
            pl.BlockSpec((E, 1), lambda i: (0, 0)),
        ],
        out_specs=(
            pl.BlockSpec((BT, E), lambda i: (i, 0)),
            pl.BlockSpec((BT, TOPK), lambda i: (i, 0)),
            pl.BlockSpec((BT, TOPK), lambda i: (i, 0)),
            pl.BlockSpec((E, TOPK, BT), lambda i: (0, 0, i)),
            pl.BlockSpec((1, 1), lambda i: (0, 0)),
        ),
        out_shape=out_shapes,
        scratch_shapes=[
            pltpu.VMEM((E, 1), jnp.float32),
            pltpu.VMEM((E, 1), jnp.float32),
        ],
    )(hidden_states, wg, wg_ema, cal2d)


def kernel(hidden_states, output_aux_losses, wg, wg_ema, cal_weights, ce_ema):
    logits, rw, sel, em, aux = _run(
        hidden_states.astype(jnp.float32), wg, wg_ema, cal_weights)
    aux_loss = aux.reshape(()) * jnp.asarray(output_aux_losses,
                                             dtype=jnp.float32)
    return (rw.astype(hidden_states.dtype), logits, aux_loss, em, sel)


# speedup vs baseline: 1.2242x; 1.0138x over previous
import functools

import jax
import jax.numpy as jnp
from jax.experimental import pallas as pl
from jax.experimental.pallas import tpu as pltpu

T = 16384
HIDDEN = 2048
E = 16
TOPK = 2
CE_ALPHA = 0.0
OVER_COMPUTE = 1.2

BT = 2048


def _router_kernel(hs_ref, wg_ref, wg_ema_ref, cal_ref,
                   logits_ref, rw_ref, sel_ref, em_ref, aux_ref,
                   me_acc, ce_acc):
    i = pl.program_id(0)
    nsteps = pl.num_programs(0)

    wg_running = 0.5 * (wg_ema_ref[...] + wg_ref[...])
    lt = jnp.dot(hs_ref[...], wg_running, preferred_element_type=jnp.float32)
    logits_ref[...] = lt

    ltt = lt.T
    cmax = jnp.max(ltt, axis=0, keepdims=True)
    ex = jnp.exp(ltt - cmax)
    den = jnp.sum(ex, axis=0, keepdims=True)
    probs_t = ex / den

    e_col = jax.lax.broadcasted_iota(jnp.int32, (E, BT), 0)
    m1 = jnp.max(probs_t, axis=0, keepdims=True)
    a1 = jnp.min(jnp.where(probs_t == m1, e_col, E), axis=0, keepdims=True)
    oh1 = e_col == a1
    probs_m = jnp.where(oh1, -1.0, probs_t)
    m2 = jnp.max(probs_m, axis=0, keepdims=True)
    a2 = jnp.min(jnp.where(probs_m == m2, e_col, E), axis=0, keepdims=True)
    oh2 = e_col == a2

    @pl.when(i == 0)
    def _init():
        me_acc[...] = jnp.zeros_like(me_acc)
        ce_acc[...] = jnp.zeros_like(ce_acc)

    mask_t = oh1.astype(jnp.float32) + oh2.astype(jnp.float32)
    me_acc[...] += jnp.sum(probs_t, axis=1, keepdims=True)
    ce_acc[...] += jnp.sum(mask_t, axis=1, keepdims=True)

    e3 = jax.lax.broadcasted_iota(jnp.int32, (E, TOPK, BT), 0)
    k3 = jax.lax.broadcasted_iota(jnp.int32, (E, TOPK, BT), 1)
    sel3 = jnp.where(k3 == 0, a1[:, None, :], a2[:, None, :])
    em_ref[...] = (e3 == sel3).astype(jnp.int32)

    rsum = m1 + m2
    pack = jnp.concatenate(
        [m1 / rsum, m2 / rsum, a1.astype(jnp.float32), a2.astype(jnp.float32),
         jnp.zeros((4, BT), jnp.float32)], axis=0)
    pack_t = pack.T
    rw_ref[...] = pack_t[:, 0:2]
    sel_ref[...] = pack_t[:, 2:4].astype(jnp.int32)

    @pl.when(i == nsteps - 1)
    def _fini():
        me = me_acc[...] / T
        ce = (ce_acc[...] / T) * cal_ref[...]
        ce_ema_new = (1.0 - CE_ALPHA) * ce
        e_idx = jax.lax.broadcasted_iota(jnp.int32, (E, 1), 0)
        hot_num = jnp.max(ce_ema_new)
        hot_exp = jnp.min(jnp.where(ce_ema_new == hot_num, e_idx, E))
        cold_num = jnp.min(ce_ema_new)
        cold_exp = jnp.min(jnp.where(ce_ema_new == cold_num, e_idx, E))
        me_hot = jnp.sum(jnp.where(e_idx == hot_exp, me, 0.0))
        me_cold = jnp.sum(jnp.where(e_idx == cold_exp, me, 0.0))
        aux = jnp.maximum(hot_num - cold_num * OVER_COMPUTE, 0.0)
        aux_ref[...] = (aux * (me_hot - me_cold))[None, None]


@jax.jit
def _run(hidden_states, wg, wg_ema, cal_weights):
    nsteps = T // BT
    cal2d = cal_weights.reshape(E, 1)
    out_shapes = (
        jax.ShapeDtypeStruct((T, E), jnp.float32),
        jax.ShapeDtypeStruct((T, TOPK), jnp.float32),
        jax.ShapeDtypeStruct((T, TOPK), jnp.int32),
        jax.ShapeDtypeStruct((E, TOPK, T), jnp.int32),
        jax.ShapeDtypeStruct((1, 1), jnp.float32),
    )
    return pl.pallas_call(
        _router_kernel,
        grid=(nsteps,),
        in_specs=[
            pl.BlockSpec((BT, HIDDEN), lambda i: (i, 0)),
            pl.BlockSpec((HIDDEN, E), lambda i: (0, 0)),
            pl.BlockSpec((HIDDEN, E), lambda i: (0, 0)),
            pl.BlockSpec((E, 1), lambda i: (0, 0)),
        ],
        out_specs=(
            pl.BlockSpec((BT, E), lambda i: (i, 0)),
            pl.BlockSpec((BT, TOPK), lambda i: (i, 0)),
            pl.BlockSpec((BT, TOPK), lambda i: (i, 0)),
            pl.BlockSpec((E, TOPK, BT), lambda i: (0, 0, i)),
            pl.BlockSpec((1, 1), lambda i: (0, 0)),
        ),
        out_shape=out_shapes,
        scratch_shapes=[
            pltpu.VMEM((E, 1), jnp.float32),
            pltpu.VMEM((E, 1), jnp.float32),
        ],
    )(hidden_states, wg, wg_ema, cal2d)


def kernel(hidden_states, output_aux_losses, wg, wg_ema, cal_weights, ce_ema):
    logits, rw, sel, em, aux = _run(
        hidden_states.astype(jnp.float32), wg, wg_ema, cal_weights)
    aux_loss = aux.reshape(()) * jnp.asarray(output_aux_losses,
                                             dtype=jnp.float32)
    return (rw.astype(hidden_states.dtype), logits, aux_loss, em, sel)
